# baseline (speedup 1.0000x reference)
import jax
import jax.numpy as jnp
import numpy as np
from jax import lax
from jax.experimental import pallas as pl
from jax.experimental.pallas import tpu as pltpu
from jax.experimental.pallas import tpu_sc as plsc

N_ROWS = 262144
N_CORES = 2
N_SUBCORES = 16
N_WORKERS = N_CORES * N_SUBCORES
ROWS_PER = N_ROWS // N_WORKERS
N_LANES = 16
CHUNK = 128
N_CHUNKS = ROWS_PER // CHUNK
UNROLL = 4
GROUPS = CHUNK // (N_LANES * UNROLL)
COL0 = 0

_IDX_NP = (np.arange(N_ROWS, dtype=np.int32) * 32 + 8).reshape(
    N_WORKERS, N_CHUNKS, CHUNK)


def _actions_bits(buf, base, lanes, accs):
    new_accs = []
    for u in range(UNROLL):
        rows = base + u * N_LANES + lanes
        c0 = jnp.zeros((N_LANES,), jnp.int32)
        mi_x = plsc.load_gather(buf, [rows, c0 + COL0])
        su_x = plsc.load_gather(buf, [rows, c0 + (COL0 + 1)])
        mi_y = plsc.load_gather(buf, [rows, c0 + (COL0 + 2)])
        su_y = plsc.load_gather(buf, [rows, c0 + (COL0 + 3)])

        dist_x = jnp.abs(su_x - mi_x)
        dist_y = jnp.abs(su_y - mi_y)
        go_down = su_y > mi_y
        go_up = su_y < mi_y
        go_right = su_x > mi_x
        go_left = su_x < mi_x
        cond_y = dist_y > 2.0
        cond_x_far = dist_x > 24.0
        cond_x_close = dist_x < 22.0

        def iv(v):
            return jnp.full((N_LANES,), v, jnp.int32)

        act_y = jnp.where(go_down, iv(5), iv(2))
        act_x_far = jnp.where(go_right, iv(3), iv(4))
        act_x_close = jnp.where(go_right, iv(4), iv(3))

        up_left = go_up & go_left
        down_right = go_down & go_right
        down_left = go_down & go_left

        adf = iv(6)
        adf = jnp.where(up_left, iv(7), adf)
        adf = jnp.where(down_right, iv(8), adf)
        adf = jnp.where(down_left, iv(9), adf)

        adc = iv(7)
        adc = jnp.where(up_left, iv(6), adc)
        adc = jnp.where(down_right, iv(9), adc)
        adc = jnp.where(down_left, iv(8), adc)

        action = iv(1)
        action = jnp.where(cond_x_close, act_x_close, action)
        action = jnp.where(cond_x_far, act_x_far, action)
        action = jnp.where(cond_y, act_y, action)
        action = jnp.where(cond_y & cond_x_far, adf, action)
        action = jnp.where(cond_y & cond_x_close, adc, action)

        new_accs.append(accs[u] | lax.shift_left(iv(1), action))
    return tuple(new_accs)


def _sc_body(ram16_hbm, idx_hbm, out_hbm, idx_v, buf0, buf1, ind_v, sidx_v,
             z_v, acc_v, out_v, shared, sem):
    cid = lax.axis_index("c")
    sid = lax.axis_index("s")
    wid = sid * N_CORES + cid
    lanes = lax.iota(jnp.int32, N_LANES)

    pltpu.sync_copy(idx_hbm.at[wid], idx_v)

    bufs = (buf0, buf1)

    def issue(c, b):
        return pltpu.async_copy(ram16_hbm.at[idx_v.at[c]], bufs[b], sem)

    def wait(b):
        pltpu.make_async_copy(ram16_hbm.at[idx_v.at[0]], bufs[b], sem).wait()

    issue(0, 0)
    issue(1, 1)

    zero = jnp.zeros((N_LANES,), jnp.int32)

    def compute(b, accs):
        for g in range(GROUPS):
            accs = _actions_bits(bufs[b], g * N_LANES * UNROLL, lanes, accs)
        return accs

    def body(i, accs):
        c = i * 2
        wait(0)
        accs = compute(0, accs)

        @pl.when(c + 2 < N_CHUNKS)
        def _():
            issue(c + 2, 0)

        wait(1)
        accs = compute(1, accs)

        @pl.when(c + 3 < N_CHUNKS)
        def _():
            issue(c + 3, 1)

        return accs

    accs = lax.fori_loop(0, N_CHUNKS // 2, body, (zero,) * UNROLL)
    bits = accs[0]
    for u in range(1, UNROLL):
        bits = bits | accs[u]

    for s in (1, 2, 4, 8):
        rot = (lanes + s) & (N_LANES - 1)
        bits = bits | bits.at[rot].get(mode="promise_in_bounds")
    ind = lax.shift_right_logical(bits, lanes) & 1
    ind_v[...] = ind.astype(jnp.float32)
    sidx_v[...] = lanes
    z_v[...] = jnp.zeros((N_LANES,), jnp.float32)

    @pl.when(sid == 0)
    def _init():
        pltpu.sync_copy(z_v, shared)

    plsc.subcore_barrier()
    pltpu.sync_copy(ind_v, shared.at[sidx_v], add=True)
    plsc.subcore_barrier()

    @pl.when(sid == 0)
    def _finalize():
        pltpu.sync_copy(shared, acc_v)
        seen = acc_v[...]
        out_v[pl.ds(0, N_LANES)] = jnp.where(
            seen > 0.0, jnp.full((N_LANES,), 1.0, jnp.float32),
            jnp.zeros((N_LANES,), jnp.float32))
        out_v[pl.ds(N_LANES, N_LANES)] = jnp.zeros((N_LANES,), jnp.float32)
        pltpu.sync_copy(out_v, out_hbm.at[cid])


@jax.jit
def kernel(ram):
    ram16 = ram.reshape(-1, 4)
    idx = jnp.asarray(_IDX_NP)
    mesh = plsc.VectorSubcoreMesh(
        core_axis_name="c", subcore_axis_name="s", num_cores=N_CORES)
    parts = pl.kernel(
        _sc_body,
        out_type=jax.ShapeDtypeStruct((N_CORES, 2 * N_LANES), jnp.float32),
        mesh=mesh,
        compiler_params=pltpu.CompilerParams(
            use_tc_tiling_on_sc=False, needs_layout_passes=False),
        scratch_types=[
            pltpu.VMEM((N_CHUNKS, CHUNK), jnp.int32),
            pltpu.VMEM((CHUNK, 4), jnp.float32),
            pltpu.VMEM((CHUNK, 4), jnp.float32),
            pltpu.VMEM((N_LANES,), jnp.float32),
            pltpu.VMEM((N_LANES,), jnp.int32),
            pltpu.VMEM((N_LANES,), jnp.float32),
            pltpu.VMEM((N_LANES,), jnp.float32),
            pltpu.VMEM((2 * N_LANES,), jnp.float32),
            pltpu.VMEM_SHARED((N_LANES,), jnp.float32),
            pltpu.SemaphoreType.DMA,
        ],
    )(ram16, idx)
    merged = jnp.maximum(parts[0], parts[1])
    return merged[:18].reshape(1, 18)


# speedup vs baseline: 200.5656x; 200.5656x over previous
import jax
import jax.numpy as jnp
import numpy as np
from jax import lax
from jax.experimental import pallas as pl
from jax.experimental.pallas import tpu as pltpu
from jax.experimental.pallas import tpu_sc as plsc

N_ROWS = 262144
N_CORES = 2
N_SUBCORES = 16
N_WORKERS = N_CORES * N_SUBCORES
ROWS_PER = N_ROWS // N_WORKERS
N_LANES = 16
CHUNK = 128
N_CHUNKS = ROWS_PER // CHUNK
UNROLL = 4
NBUF = 4
GROUPS = CHUNK // (N_LANES * UNROLL)
COL0 = 2

_IDX_NP = (np.arange(N_ROWS, dtype=np.int32) * 8 + 2).reshape(
    N_WORKERS, N_CHUNKS, CHUNK)


def _actions_bits(buf, base, lanes, accs):
    new_accs = []
    for u in range(UNROLL):
        rows = base + u * N_LANES + lanes
        c0 = jnp.zeros((N_LANES,), jnp.int32)
        mi_x = plsc.load_gather(buf, [rows, c0 + COL0])
        su_x = plsc.load_gather(buf, [rows, c0 + (COL0 + 1)])
        mi_y = plsc.load_gather(buf, [rows, c0 + (COL0 + 2)])
        su_y = plsc.load_gather(buf, [rows, c0 + (COL0 + 3)])

        dist_x = jnp.abs(su_x - mi_x)
        dist_y = jnp.abs(su_y - mi_y)
        go_down = su_y > mi_y
        go_up = su_y < mi_y
        go_right = su_x > mi_x
        go_left = su_x < mi_x
        cond_y = dist_y > 2.0
        cond_x_far = dist_x > 24.0
        cond_x_close = dist_x < 22.0

        def iv(v):
            return jnp.full((N_LANES,), v, jnp.int32)

        act_y = jnp.where(go_down, iv(5), iv(2))
        act_x_far = jnp.where(go_right, iv(3), iv(4))
        act_x_close = jnp.where(go_right, iv(4), iv(3))

        up_left = go_up & go_left
        down_right = go_down & go_right
        down_left = go_down & go_left

        adf = iv(6)
        adf = jnp.where(up_left, iv(7), adf)
        adf = jnp.where(down_right, iv(8), adf)
        adf = jnp.where(down_left, iv(9), adf)

        adc = iv(7)
        adc = jnp.where(up_left, iv(6), adc)
        adc = jnp.where(down_right, iv(9), adc)
        adc = jnp.where(down_left, iv(8), adc)

        action = iv(1)
        action = jnp.where(cond_x_close, act_x_close, action)
        action = jnp.where(cond_x_far, act_x_far, action)
        action = jnp.where(cond_y, act_y, action)
        action = jnp.where(cond_y & cond_x_far, adf, action)
        action = jnp.where(cond_y & cond_x_close, adc, action)

        new_accs.append(accs[u] | lax.shift_left(iv(1), action))
    return tuple(new_accs)


def _sc_body(ram16_hbm, idx_hbm, out_hbm, idx_v, buf0, buf1, buf2, buf3,
             ind_v, sidx_v, z_v, acc_v, out_v, shared, sem):
    cid = lax.axis_index("c")
    sid = lax.axis_index("s")
    wid = sid * N_CORES + cid
    lanes = lax.iota(jnp.int32, N_LANES)

    pltpu.sync_copy(idx_hbm.at[wid], idx_v)

    bufs = (buf0, buf1, buf2, buf3)

    def issue(c, b):
        return pltpu.async_copy(ram16_hbm.at[idx_v.at[c]], bufs[b], sem)

    def wait(b):
        pltpu.make_async_copy(ram16_hbm.at[idx_v.at[0]], bufs[b], sem).wait()

    for b in range(NBUF):
        issue(b, b)

    zero = jnp.zeros((N_LANES,), jnp.int32)

    def compute(b, accs):
        for g in range(GROUPS):
            accs = _actions_bits(bufs[b], g * N_LANES * UNROLL, lanes, accs)
        return accs

    def body(i, accs):
        c = i * NBUF
        for b in range(NBUF):
            wait(b)
            accs = compute(b, accs)

            @pl.when(c + b + NBUF < N_CHUNKS)
            def _():
                issue(c + b + NBUF, b)

        return accs

    accs = lax.fori_loop(0, N_CHUNKS // NBUF, body, (zero,) * UNROLL)
    bits = accs[0]
    for u in range(1, UNROLL):
        bits = bits | accs[u]

    for s in (1, 2, 4, 8):
        rot = (lanes + s) & (N_LANES - 1)
        bits = bits | bits.at[rot].get(mode="promise_in_bounds")
    ind = lax.shift_right_logical(bits, lanes) & 1
    ind_v[...] = ind.astype(jnp.float32)
    sidx_v[...] = lanes
    z_v[...] = jnp.zeros((N_LANES,), jnp.float32)

    @pl.when(sid == 0)
    def _init():
        pltpu.sync_copy(z_v, shared)

    plsc.subcore_barrier()
    pltpu.sync_copy(ind_v, shared.at[sidx_v], add=True)
    plsc.subcore_barrier()

    @pl.when(sid == 0)
    def _finalize():
        pltpu.sync_copy(shared, acc_v)
        seen = acc_v[...]
        out_v[pl.ds(0, N_LANES)] = jnp.where(
            seen > 0.0, jnp.full((N_LANES,), 1.0, jnp.float32),
            jnp.zeros((N_LANES,), jnp.float32))
        out_v[pl.ds(N_LANES, N_LANES)] = jnp.zeros((N_LANES,), jnp.float32)
        pltpu.sync_copy(out_v, out_hbm.at[cid])


@jax.jit
def kernel(ram):
    ram16 = ram.reshape(-1, 16)
    idx = jnp.asarray(_IDX_NP)
    mesh = plsc.VectorSubcoreMesh(
        core_axis_name="c", subcore_axis_name="s", num_cores=N_CORES)
    parts = pl.kernel(
        _sc_body,
        out_type=jax.ShapeDtypeStruct((N_CORES, 2 * N_LANES), jnp.float32),
        mesh=mesh,
        compiler_params=pltpu.CompilerParams(
            use_tc_tiling_on_sc=False, needs_layout_passes=False),
        scratch_types=[
            pltpu.VMEM((N_CHUNKS, CHUNK), jnp.int32),
            pltpu.VMEM((CHUNK, 16), jnp.float32),
            pltpu.VMEM((CHUNK, 16), jnp.float32),
            pltpu.VMEM((CHUNK, 16), jnp.float32),
            pltpu.VMEM((CHUNK, 16), jnp.float32),
            pltpu.VMEM((N_LANES,), jnp.float32),
            pltpu.VMEM((N_LANES,), jnp.int32),
            pltpu.VMEM((N_LANES,), jnp.float32),
            pltpu.VMEM((N_LANES,), jnp.float32),
            pltpu.VMEM((2 * N_LANES,), jnp.float32),
            pltpu.VMEM_SHARED((N_LANES,), jnp.float32),
            pltpu.SemaphoreType.DMA,
        ],
    )(ram16, idx)
    merged = jnp.maximum(parts[0], parts[1])
    return merged[:18].reshape(1, 18)


# speedup vs baseline: 233.3359x; 1.1634x over previous
import jax
import jax.numpy as jnp
import numpy as np
from jax import lax
from jax.experimental import pallas as pl
from jax.experimental.pallas import tpu as pltpu
from jax.experimental.pallas import tpu_sc as plsc

N_ROWS = 262144
N_CORES = 2
N_SUBCORES = 16
N_WORKERS = N_CORES * N_SUBCORES
ROWS_PER = N_ROWS // N_WORKERS
N_LANES = 16
CHUNK = 128
N_CHUNKS = ROWS_PER // CHUNK
UNROLL = 4
NBUF = 8
GROUPS = CHUNK // (N_LANES * UNROLL)
COL0 = 2

_IDX_NP = (np.arange(N_ROWS, dtype=np.int32) * 8 + 2).reshape(
    N_WORKERS, N_CHUNKS, CHUNK)


def _actions_bits(buf, base, lanes, accs):
    new_accs = []
    for u in range(UNROLL):
        rows = base + u * N_LANES + lanes
        c0 = jnp.zeros((N_LANES,), jnp.int32)
        mi_x = plsc.load_gather(buf, [rows, c0 + COL0])
        su_x = plsc.load_gather(buf, [rows, c0 + (COL0 + 1)])
        mi_y = plsc.load_gather(buf, [rows, c0 + (COL0 + 2)])
        su_y = plsc.load_gather(buf, [rows, c0 + (COL0 + 3)])

        dist_x = jnp.abs(su_x - mi_x)
        dist_y = jnp.abs(su_y - mi_y)
        go_down = su_y > mi_y
        go_up = su_y < mi_y
        go_right = su_x > mi_x
        go_left = su_x < mi_x
        cond_y = dist_y > 2.0
        cond_x_far = dist_x > 24.0
        cond_x_close = dist_x < 22.0

        def iv(v):
            return jnp.full((N_LANES,), v, jnp.int32)

        act_y = jnp.where(go_down, iv(5), iv(2))
        act_x_far = jnp.where(go_right, iv(3), iv(4))
        act_x_close = jnp.where(go_right, iv(4), iv(3))

        up_left = go_up & go_left
        down_right = go_down & go_right
        down_left = go_down & go_left

        adf = iv(6)
        adf = jnp.where(up_left, iv(7), adf)
        adf = jnp.where(down_right, iv(8), adf)
        adf = jnp.where(down_left, iv(9), adf)

        adc = iv(7)
        adc = jnp.where(up_left, iv(6), adc)
        adc = jnp.where(down_right, iv(9), adc)
        adc = jnp.where(down_left, iv(8), adc)

        action = iv(1)
        action = jnp.where(cond_x_close, act_x_close, action)
        action = jnp.where(cond_x_far, act_x_far, action)
        action = jnp.where(cond_y, act_y, action)
        action = jnp.where(cond_y & cond_x_far, adf, action)
        action = jnp.where(cond_y & cond_x_close, adc, action)

        new_accs.append(accs[u] | lax.shift_left(iv(1), action))
    return tuple(new_accs)


def _sc_body(ram16_hbm, idx_hbm, out_hbm, idx_v, buf0, buf1, buf2, buf3,
             buf4, buf5, buf6, buf7, ind_v, sidx_v, z_v, acc_v, out_v,
             shared, sem):
    cid = lax.axis_index("c")
    sid = lax.axis_index("s")
    wid = sid * N_CORES + cid
    lanes = lax.iota(jnp.int32, N_LANES)

    pltpu.sync_copy(idx_hbm.at[wid], idx_v)

    bufs = (buf0, buf1, buf2, buf3, buf4, buf5, buf6, buf7)

    def issue(c, b):
        return pltpu.async_copy(ram16_hbm.at[idx_v.at[c]], bufs[b], sem)

    def wait(b):
        pltpu.make_async_copy(ram16_hbm.at[idx_v.at[0]], bufs[b], sem).wait()

    for b in range(NBUF):
        issue(b, b)

    zero = jnp.zeros((N_LANES,), jnp.int32)

    def compute(b, accs):
        for g in range(GROUPS):
            accs = _actions_bits(bufs[b], g * N_LANES * UNROLL, lanes, accs)
        return accs

    def body(i, accs):
        c = i * NBUF
        for b in range(NBUF):
            wait(b)
            accs = compute(b, accs)

            @pl.when(c + b + NBUF < N_CHUNKS)
            def _():
                issue(c + b + NBUF, b)

        return accs

    accs = lax.fori_loop(0, N_CHUNKS // NBUF, body, (zero,) * UNROLL)
    bits = accs[0]
    for u in range(1, UNROLL):
        bits = bits | accs[u]

    for s in (1, 2, 4, 8):
        rot = (lanes + s) & (N_LANES - 1)
        bits = bits | bits.at[rot].get(mode="promise_in_bounds")
    ind = lax.shift_right_logical(bits, lanes) & 1
    ind_v[...] = ind.astype(jnp.float32)
    sidx_v[...] = lanes
    z_v[...] = jnp.zeros((N_LANES,), jnp.float32)

    @pl.when(sid == 0)
    def _init():
        pltpu.sync_copy(z_v, shared)

    plsc.subcore_barrier()
    pltpu.sync_copy(ind_v, shared.at[sidx_v], add=True)
    plsc.subcore_barrier()

    @pl.when(sid == 0)
    def _finalize():
        pltpu.sync_copy(shared, acc_v)
        seen = acc_v[...]
        out_v[pl.ds(0, N_LANES)] = jnp.where(
            seen > 0.0, jnp.full((N_LANES,), 1.0, jnp.float32),
            jnp.zeros((N_LANES,), jnp.float32))
        out_v[pl.ds(N_LANES, N_LANES)] = jnp.zeros((N_LANES,), jnp.float32)
        pltpu.sync_copy(out_v, out_hbm.at[cid])


@jax.jit
def kernel(ram):
    ram16 = ram.reshape(-1, 16)
    idx = jnp.asarray(_IDX_NP)
    mesh = plsc.VectorSubcoreMesh(
        core_axis_name="c", subcore_axis_name="s", num_cores=N_CORES)
    parts = pl.kernel(
        _sc_body,
        out_type=jax.ShapeDtypeStruct((N_CORES, 2 * N_LANES), jnp.float32),
        mesh=mesh,
        compiler_params=pltpu.CompilerParams(
            use_tc_tiling_on_sc=False, needs_layout_passes=False),
        scratch_types=[
            pltpu.VMEM((N_CHUNKS, CHUNK), jnp.int32),
            pltpu.VMEM((CHUNK, 16), jnp.float32),
            pltpu.VMEM((CHUNK, 16), jnp.float32),
            pltpu.VMEM((CHUNK, 16), jnp.float32),
            pltpu.VMEM((CHUNK, 16), jnp.float32),
            pltpu.VMEM((CHUNK, 16), jnp.float32),
            pltpu.VMEM((CHUNK, 16), jnp.float32),
            pltpu.VMEM((CHUNK, 16), jnp.float32),
            pltpu.VMEM((CHUNK, 16), jnp.float32),
            pltpu.VMEM((N_LANES,), jnp.float32),
            pltpu.VMEM((N_LANES,), jnp.int32),
            pltpu.VMEM((N_LANES,), jnp.float32),
            pltpu.VMEM((N_LANES,), jnp.float32),
            pltpu.VMEM((2 * N_LANES,), jnp.float32),
            pltpu.VMEM_SHARED((N_LANES,), jnp.float32),
            pltpu.SemaphoreType.DMA,
        ],
    )(ram16, idx)
    merged = jnp.maximum(parts[0], parts[1])
    return merged[:18].reshape(1, 18)


# speedup vs baseline: 242.2967x; 1.0384x over previous
import jax
import jax.numpy as jnp
import numpy as np
from jax import lax
from jax.experimental import pallas as pl
from jax.experimental.pallas import tpu as pltpu
from jax.experimental.pallas import tpu_sc as plsc

N_ROWS = 262144
N_CORES = 2
N_SUBCORES = 16
N_WORKERS = N_CORES * N_SUBCORES
ROWS_PER = N_ROWS // N_WORKERS
N_LANES = 16
CHUNK = 128
N_CHUNKS = ROWS_PER // CHUNK
UNROLL = 4
NBUF = 16
GROUPS = CHUNK // (N_LANES * UNROLL)
COL0 = 2

_IDX_NP = (np.arange(N_ROWS, dtype=np.int32) * 8 + 2).reshape(
    N_WORKERS, N_CHUNKS, CHUNK)


def _actions_bits(buf, base, lanes, accs):
    new_accs = []
    for u in range(UNROLL):
        rows = base + u * N_LANES + lanes
        c0 = jnp.zeros((N_LANES,), jnp.int32)
        mi_x = plsc.load_gather(buf, [rows, c0 + COL0])
        su_x = plsc.load_gather(buf, [rows, c0 + (COL0 + 1)])
        mi_y = plsc.load_gather(buf, [rows, c0 + (COL0 + 2)])
        su_y = plsc.load_gather(buf, [rows, c0 + (COL0 + 3)])

        dist_x = jnp.abs(su_x - mi_x)
        dist_y = jnp.abs(su_y - mi_y)
        go_down = su_y > mi_y
        go_up = su_y < mi_y
        go_right = su_x > mi_x
        go_left = su_x < mi_x
        cond_y = dist_y > 2.0
        cond_x_far = dist_x > 24.0
        cond_x_close = dist_x < 22.0

        def iv(v):
            return jnp.full((N_LANES,), v, jnp.int32)

        act_y = jnp.where(go_down, iv(5), iv(2))
        act_x_far = jnp.where(go_right, iv(3), iv(4))
        act_x_close = jnp.where(go_right, iv(4), iv(3))

        up_left = go_up & go_left
        down_right = go_down & go_right
        down_left = go_down & go_left

        adf = iv(6)
        adf = jnp.where(up_left, iv(7), adf)
        adf = jnp.where(down_right, iv(8), adf)
        adf = jnp.where(down_left, iv(9), adf)

        adc = iv(7)
        adc = jnp.where(up_left, iv(6), adc)
        adc = jnp.where(down_right, iv(9), adc)
        adc = jnp.where(down_left, iv(8), adc)

        action = iv(1)
        action = jnp.where(cond_x_close, act_x_close, action)
        action = jnp.where(cond_x_far, act_x_far, action)
        action = jnp.where(cond_y, act_y, action)
        action = jnp.where(cond_y & cond_x_far, adf, action)
        action = jnp.where(cond_y & cond_x_close, adc, action)

        new_accs.append(accs[u] | lax.shift_left(iv(1), action))
    return tuple(new_accs)


def _sc_body(ram16_hbm, idx_hbm, out_hbm, idx_v, buf0, buf1, buf2, buf3,
             buf4, buf5, buf6, buf7, buf8, buf9, buf10, buf11, buf12, buf13,
             buf14, buf15, ind_v, sidx_v, z_v, acc_v, out_v,
             shared, sem):
    cid = lax.axis_index("c")
    sid = lax.axis_index("s")
    wid = sid * N_CORES + cid
    lanes = lax.iota(jnp.int32, N_LANES)

    pltpu.sync_copy(idx_hbm.at[wid], idx_v)

    bufs = (buf0, buf1, buf2, buf3, buf4, buf5, buf6, buf7,
            buf8, buf9, buf10, buf11, buf12, buf13, buf14, buf15)

    def issue(c, b):
        return pltpu.async_copy(ram16_hbm.at[idx_v.at[c]], bufs[b], sem)

    def wait(b):
        pltpu.make_async_copy(ram16_hbm.at[idx_v.at[0]], bufs[b], sem).wait()

    for b in range(NBUF):
        issue(b, b)

    zero = jnp.zeros((N_LANES,), jnp.int32)

    def compute(b, accs):
        for g in range(GROUPS):
            accs = _actions_bits(bufs[b], g * N_LANES * UNROLL, lanes, accs)
        return accs

    def body(i, accs):
        c = i * NBUF
        for b in range(NBUF):
            wait(b)
            accs = compute(b, accs)

            @pl.when(c + b + NBUF < N_CHUNKS)
            def _():
                issue(c + b + NBUF, b)

        return accs

    accs = lax.fori_loop(0, N_CHUNKS // NBUF, body, (zero,) * UNROLL)
    bits = accs[0]
    for u in range(1, UNROLL):
        bits = bits | accs[u]

    for s in (1, 2, 4, 8):
        rot = (lanes + s) & (N_LANES - 1)
        bits = bits | bits.at[rot].get(mode="promise_in_bounds")
    ind = lax.shift_right_logical(bits, lanes) & 1
    ind_v[...] = ind.astype(jnp.float32)
    sidx_v[...] = lanes
    z_v[...] = jnp.zeros((N_LANES,), jnp.float32)

    @pl.when(sid == 0)
    def _init():
        pltpu.sync_copy(z_v, shared)

    plsc.subcore_barrier()
    pltpu.sync_copy(ind_v, shared.at[sidx_v], add=True)
    plsc.subcore_barrier()

    @pl.when(sid == 0)
    def _finalize():
        pltpu.sync_copy(shared, acc_v)
        seen = acc_v[...]
        out_v[pl.ds(0, N_LANES)] = jnp.where(
            seen > 0.0, jnp.full((N_LANES,), 1.0, jnp.float32),
            jnp.zeros((N_LANES,), jnp.float32))
        out_v[pl.ds(N_LANES, N_LANES)] = jnp.zeros((N_LANES,), jnp.float32)
        pltpu.sync_copy(out_v, out_hbm.at[cid])


@jax.jit
def kernel(ram):
    ram16 = ram.reshape(-1, 16)
    idx = jnp.asarray(_IDX_NP)
    mesh = plsc.VectorSubcoreMesh(
        core_axis_name="c", subcore_axis_name="s", num_cores=N_CORES)
    parts = pl.kernel(
        _sc_body,
        out_type=jax.ShapeDtypeStruct((N_CORES, 2 * N_LANES), jnp.float32),
        mesh=mesh,
        compiler_params=pltpu.CompilerParams(
            use_tc_tiling_on_sc=False, needs_layout_passes=False),
        scratch_types=[
            pltpu.VMEM((N_CHUNKS, CHUNK), jnp.int32),
            pltpu.VMEM((CHUNK, 16), jnp.float32),
            pltpu.VMEM((CHUNK, 16), jnp.float32),
            pltpu.VMEM((CHUNK, 16), jnp.float32),
            pltpu.VMEM((CHUNK, 16), jnp.float32),
            pltpu.VMEM((CHUNK, 16), jnp.float32),
            pltpu.VMEM((CHUNK, 16), jnp.float32),
            pltpu.VMEM((CHUNK, 16), jnp.float32),
            pltpu.VMEM((CHUNK, 16), jnp.float32),
            pltpu.VMEM((CHUNK, 16), jnp.float32),
            pltpu.VMEM((CHUNK, 16), jnp.float32),
            pltpu.VMEM((CHUNK, 16), jnp.float32),
            pltpu.VMEM((CHUNK, 16), jnp.float32),
            pltpu.VMEM((CHUNK, 16), jnp.float32),
            pltpu.VMEM((CHUNK, 16), jnp.float32),
            pltpu.VMEM((CHUNK, 16), jnp.float32),
            pltpu.VMEM((CHUNK, 16), jnp.float32),
            pltpu.VMEM((N_LANES,), jnp.float32),
            pltpu.VMEM((N_LANES,), jnp.int32),
            pltpu.VMEM((N_LANES,), jnp.float32),
            pltpu.VMEM((N_LANES,), jnp.float32),
            pltpu.VMEM((2 * N_LANES,), jnp.float32),
            pltpu.VMEM_SHARED((N_LANES,), jnp.float32),
            pltpu.SemaphoreType.DMA,
        ],
    )(ram16, idx)
    merged = jnp.maximum(parts[0], parts[1])
    return merged[:18].reshape(1, 18)


# speedup vs baseline: 243.4591x; 1.0048x over previous
import jax
import jax.numpy as jnp
import numpy as np
from jax import lax
from jax.experimental import pallas as pl
from jax.experimental.pallas import tpu as pltpu
from jax.experimental.pallas import tpu_sc as plsc

N_ROWS = 262144
N_CORES = 2
N_SUBCORES = 16
N_WORKERS = N_CORES * N_SUBCORES
ROWS_PER = N_ROWS // N_WORKERS
N_LANES = 16
CHUNK = 128
N_CHUNKS = ROWS_PER // CHUNK
UNROLL = 4
NBUF = 16
GROUPS = CHUNK // (N_LANES * UNROLL)
COL0 = 2

_IDX_NP = (np.arange(N_ROWS, dtype=np.int32) * 8 + 2).reshape(
    N_WORKERS, N_CHUNKS, CHUNK)


def _actions_bits(buf, base, lanes, accs):
    new_accs = []
    for u in range(UNROLL):
        rows = base + u * N_LANES + lanes
        c0 = jnp.zeros((N_LANES,), jnp.int32)
        mi_x = plsc.load_gather(buf, [rows, c0 + COL0])
        su_x = plsc.load_gather(buf, [rows, c0 + (COL0 + 1)])
        mi_y = plsc.load_gather(buf, [rows, c0 + (COL0 + 2)])
        su_y = plsc.load_gather(buf, [rows, c0 + (COL0 + 3)])

        dist_x = jnp.abs(su_x - mi_x)
        dist_y = jnp.abs(su_y - mi_y)
        go_down = su_y > mi_y
        go_up = su_y < mi_y
        go_right = su_x > mi_x
        go_left = su_x < mi_x
        cond_y = dist_y > 2.0
        cond_x_far = dist_x > 24.0
        cond_x_close = dist_x < 22.0

        def iv(v):
            return jnp.full((N_LANES,), v, jnp.int32)

        act_y = jnp.where(go_down, iv(5), iv(2))
        act_x_far = jnp.where(go_right, iv(3), iv(4))
        act_x_close = jnp.where(go_right, iv(4), iv(3))

        up_left = go_up & go_left
        down_right = go_down & go_right
        down_left = go_down & go_left

        adf = iv(6)
        adf = jnp.where(up_left, iv(7), adf)
        adf = jnp.where(down_right, iv(8), adf)
        adf = jnp.where(down_left, iv(9), adf)

        adc = iv(7)
        adc = jnp.where(up_left, iv(6), adc)
        adc = jnp.where(down_right, iv(9), adc)
        adc = jnp.where(down_left, iv(8), adc)

        action = iv(1)
        action = jnp.where(cond_x_close, act_x_close, action)
        action = jnp.where(cond_x_far, act_x_far, action)
        action = jnp.where(cond_y, act_y, action)
        action = jnp.where(cond_y & cond_x_far, adf, action)
        action = jnp.where(cond_y & cond_x_close, adc, action)

        new_accs.append(accs[u] | lax.shift_left(iv(1), action))
    return tuple(new_accs)


def _sc_body(ram16_hbm, idx_hbm, out_hbm, idx_v, buf0, buf1, buf2, buf3,
             buf4, buf5, buf6, buf7, buf8, buf9, buf10, buf11, buf12, buf13,
             buf14, buf15, ind_v, sidx_v, z_v, acc_v, out_v,
             shared, sem):
    cid = lax.axis_index("c")
    sid = lax.axis_index("s")
    wid = sid * N_CORES + cid
    lanes = lax.iota(jnp.int32, N_LANES)

    pltpu.sync_copy(idx_hbm.at[wid], idx_v)

    bufs = (buf0, buf1, buf2, buf3, buf4, buf5, buf6, buf7,
            buf8, buf9, buf10, buf11, buf12, buf13, buf14, buf15)

    def issue(c, b):
        return pltpu.async_copy(ram16_hbm.at[idx_v.at[c]], bufs[b], sem)

    def wait(b):
        pltpu.make_async_copy(ram16_hbm.at[idx_v.at[0]], bufs[b], sem).wait()

    for b in range(NBUF):
        issue(b, b)

    zero = jnp.zeros((N_LANES,), jnp.int32)

    def compute(b, accs):
        for g in range(GROUPS):
            accs = _actions_bits(bufs[b], g * N_LANES * UNROLL, lanes, accs)
        return accs

    def body(i, accs):
        c = i * NBUF
        for b in range(NBUF):
            wait(b)
            accs = compute(b, accs)

            @pl.when(c + b + NBUF < N_CHUNKS)
            def _():
                issue(c + b + NBUF, b)

        return accs

    accs = lax.fori_loop(0, N_CHUNKS // NBUF, body, (zero,) * UNROLL)
    bits = accs[0]
    for u in range(1, UNROLL):
        bits = bits | accs[u]

    for s in (1, 2, 4, 8):
        rot = (lanes + s) & (N_LANES - 1)
        bits = bits | bits.at[rot].get(mode="promise_in_bounds")
    ind = lax.shift_right_logical(bits, lanes) & 1
    ind_v[...] = ind.astype(jnp.float32)
    sidx_v[...] = lanes
    z_v[...] = jnp.zeros((N_LANES,), jnp.float32)

    @pl.when(sid == 0)
    def _init():
        pltpu.sync_copy(z_v, shared)

    plsc.subcore_barrier()
    pltpu.sync_copy(ind_v, shared.at[sidx_v], add=True)
    plsc.subcore_barrier()

    @pl.when(sid == 0)
    def _finalize():
        pltpu.sync_copy(shared, acc_v)
        seen = acc_v[...]
        out_v[pl.ds(0, N_LANES)] = jnp.where(
            seen > 0.0, jnp.full((N_LANES,), 1.0, jnp.float32),
            jnp.zeros((N_LANES,), jnp.float32))
        out_v[pl.ds(N_LANES, N_LANES)] = jnp.zeros((N_LANES,), jnp.float32)
        pltpu.sync_copy(out_v, out_hbm.at[cid])


@jax.jit
def kernel(ram):
    ram16 = ram.reshape(-1, 16)
    idx = jnp.asarray(_IDX_NP)
    mesh = plsc.VectorSubcoreMesh(
        core_axis_name="c", subcore_axis_name="s", num_cores=N_CORES)
    parts = pl.kernel(
        _sc_body,
        out_type=jax.ShapeDtypeStruct((N_CORES, 2 * N_LANES), jnp.float32),
        mesh=mesh,
        compiler_params=pltpu.CompilerParams(
            use_tc_tiling_on_sc=False, needs_layout_passes=False,
            disable_bounds_checks=True, disable_semaphore_checks=True,
            skip_device_barrier=True),
        scratch_types=[
            pltpu.VMEM((N_CHUNKS, CHUNK), jnp.int32),
            pltpu.VMEM((CHUNK, 16), jnp.float32),
            pltpu.VMEM((CHUNK, 16), jnp.float32),
            pltpu.VMEM((CHUNK, 16), jnp.float32),
            pltpu.VMEM((CHUNK, 16), jnp.float32),
            pltpu.VMEM((CHUNK, 16), jnp.float32),
            pltpu.VMEM((CHUNK, 16), jnp.float32),
            pltpu.VMEM((CHUNK, 16), jnp.float32),
            pltpu.VMEM((CHUNK, 16), jnp.float32),
            pltpu.VMEM((CHUNK, 16), jnp.float32),
            pltpu.VMEM((CHUNK, 16), jnp.float32),
            pltpu.VMEM((CHUNK, 16), jnp.float32),
            pltpu.VMEM((CHUNK, 16), jnp.float32),
            pltpu.VMEM((CHUNK, 16), jnp.float32),
            pltpu.VMEM((CHUNK, 16), jnp.float32),
            pltpu.VMEM((CHUNK, 16), jnp.float32),
            pltpu.VMEM((CHUNK, 16), jnp.float32),
            pltpu.VMEM((N_LANES,), jnp.float32),
            pltpu.VMEM((N_LANES,), jnp.int32),
            pltpu.VMEM((N_LANES,), jnp.float32),
            pltpu.VMEM((N_LANES,), jnp.float32),
            pltpu.VMEM((2 * N_LANES,), jnp.float32),
            pltpu.VMEM_SHARED((N_LANES,), jnp.float32),
            pltpu.SemaphoreType.DMA,
        ],
    )(ram16, idx)
    merged = jnp.maximum(parts[0], parts[1])
    return merged[:18].reshape(1, 18)


# speedup vs baseline: 246.6008x; 1.0129x over previous
import jax
import jax.numpy as jnp
import numpy as np
from jax import lax
from jax.experimental import pallas as pl
from jax.experimental.pallas import tpu as pltpu
from jax.experimental.pallas import tpu_sc as plsc

N_ROWS = 262144
N_CORES = 2
N_SUBCORES = 16
N_WORKERS = N_CORES * N_SUBCORES
ROWS_PER = N_ROWS // N_WORKERS
N_LANES = 16
CHUNK = 128
N_CHUNKS = ROWS_PER // CHUNK
NBUF = 16
UNROLL = 4
GROUPS = CHUNK // (N_LANES * UNROLL)
COL0 = 2

_IDX_NP = (np.arange(N_ROWS, dtype=np.int32) * 8 + 2).reshape(
    N_WORKERS, N_CHUNKS, CHUNK)


def _actions_bits(buf, base, lanes, accs):
    new_accs = []
    for u in range(UNROLL):
        rows = base + u * N_LANES + lanes
        c0 = jnp.zeros((N_LANES,), jnp.int32)
        mi_x = plsc.load_gather(buf, [rows, c0 + COL0])
        su_x = plsc.load_gather(buf, [rows, c0 + (COL0 + 1)])
        mi_y = plsc.load_gather(buf, [rows, c0 + (COL0 + 2)])
        su_y = plsc.load_gather(buf, [rows, c0 + (COL0 + 3)])

        dist_x = jnp.abs(su_x - mi_x)
        dist_y = jnp.abs(su_y - mi_y)
        go_down = su_y > mi_y
        go_up = su_y < mi_y
        go_right = su_x > mi_x
        go_left = su_x < mi_x
        cond_y = dist_y > 2.0
        cond_x_far = dist_x > 24.0
        cond_x_close = dist_x < 22.0

        def iv(v):
            return jnp.full((N_LANES,), v, jnp.int32)

        act_y = jnp.where(go_down, iv(5), iv(2))
        act_x_far = jnp.where(go_right, iv(3), iv(4))
        act_x_close = jnp.where(go_right, iv(4), iv(3))

        adf = iv(6)
        adf = jnp.where(go_up & go_left, iv(7), adf)
        adf = jnp.where(go_down & go_right, iv(8), adf)
        adf = jnp.where(go_down & go_left, iv(9), adf)
        adc = adf ^ 1

        action = jnp.where(
            cond_y,
            jnp.where(cond_x_close, adc, jnp.where(cond_x_far, adf, act_y)),
            jnp.where(cond_x_far, act_x_far,
                      jnp.where(cond_x_close, act_x_close, iv(1))))

        new_accs.append(accs[u] | lax.shift_left(iv(1), action))
    return tuple(new_accs)


def _sc_body(ram16_hbm, idx_hbm, out_hbm, idx_v, buf0, buf1, buf2, buf3,
             buf4, buf5, buf6, buf7, buf8, buf9, buf10, buf11, buf12, buf13,
             buf14, buf15, ind_v, sidx_v, z_v, acc_v, out_v, shared, sem):
    cid = lax.axis_index("c")
    sid = lax.axis_index("s")
    wid = sid * N_CORES + cid
    lanes = lax.iota(jnp.int32, N_LANES)

    pltpu.sync_copy(idx_hbm.at[wid], idx_v)

    bufs = (buf0, buf1, buf2, buf3, buf4, buf5, buf6, buf7,
            buf8, buf9, buf10, buf11, buf12, buf13, buf14, buf15)

    def issue(c, b):
        return pltpu.async_copy(ram16_hbm.at[idx_v.at[c]], bufs[b], sem)

    def wait(b):
        pltpu.make_async_copy(ram16_hbm.at[idx_v.at[0]], bufs[b], sem).wait()

    for b in range(NBUF):
        issue(b, b)

    zero = jnp.zeros((N_LANES,), jnp.int32)

    def compute(b, accs):
        for g in range(GROUPS):
            accs = _actions_bits(bufs[b], g * N_LANES * UNROLL, lanes, accs)
        return accs

    def body(i, accs):
        c = i * NBUF
        for b in range(NBUF):
            wait(b)
            accs = compute(b, accs)

            @pl.when(c + b + NBUF < N_CHUNKS)
            def _():
                issue(c + b + NBUF, b)

        return accs

    accs = lax.fori_loop(0, N_CHUNKS // NBUF, body, (zero,) * UNROLL)
    bits = accs[0]
    for u in range(1, UNROLL):
        bits = bits | accs[u]

    for s in (1, 2, 4, 8):
        rot = (lanes + s) & (N_LANES - 1)
        bits = bits | bits.at[rot].get(mode="promise_in_bounds")
    ind = lax.shift_right_logical(bits, lanes) & 1
    ind_v[...] = ind.astype(jnp.float32)
    sidx_v[...] = lanes
    z_v[...] = jnp.zeros((N_LANES,), jnp.float32)

    @pl.when(sid == 0)
    def _init():
        pltpu.sync_copy(z_v, shared)

    plsc.subcore_barrier()
    pltpu.sync_copy(ind_v, shared.at[sidx_v], add=True)
    plsc.subcore_barrier()

    @pl.when(sid == 0)
    def _finalize():
        pltpu.sync_copy(shared, acc_v)
        seen = acc_v[...]
        out_v[pl.ds(0, N_LANES)] = jnp.where(
            seen > 0.0, jnp.full((N_LANES,), 1.0, jnp.float32),
            jnp.zeros((N_LANES,), jnp.float32))
        out_v[pl.ds(N_LANES, N_LANES)] = jnp.zeros((N_LANES,), jnp.float32)
        pltpu.sync_copy(out_v, out_hbm.at[cid])


@jax.jit
def kernel(ram):
    ram16 = ram.reshape(-1, 16)
    idx = jnp.asarray(_IDX_NP)
    mesh = plsc.VectorSubcoreMesh(
        core_axis_name="c", subcore_axis_name="s", num_cores=N_CORES)
    parts = pl.kernel(
        _sc_body,
        out_type=jax.ShapeDtypeStruct((N_CORES, 2 * N_LANES), jnp.float32),
        mesh=mesh,
        compiler_params=pltpu.CompilerParams(
            use_tc_tiling_on_sc=False, needs_layout_passes=False,
            disable_bounds_checks=True, disable_semaphore_checks=True,
            skip_device_barrier=True),
        scratch_types=[
            pltpu.VMEM((N_CHUNKS, CHUNK), jnp.int32),
            pltpu.VMEM((CHUNK, 16), jnp.float32),
            pltpu.VMEM((CHUNK, 16), jnp.float32),
            pltpu.VMEM((CHUNK, 16), jnp.float32),
            pltpu.VMEM((CHUNK, 16), jnp.float32),
            pltpu.VMEM((CHUNK, 16), jnp.float32),
            pltpu.VMEM((CHUNK, 16), jnp.float32),
            pltpu.VMEM((CHUNK, 16), jnp.float32),
            pltpu.VMEM((CHUNK, 16), jnp.float32),
            pltpu.VMEM((CHUNK, 16), jnp.float32),
            pltpu.VMEM((CHUNK, 16), jnp.float32),
            pltpu.VMEM((CHUNK, 16), jnp.float32),
            pltpu.VMEM((CHUNK, 16), jnp.float32),
            pltpu.VMEM((CHUNK, 16), jnp.float32),
            pltpu.VMEM((CHUNK, 16), jnp.float32),
            pltpu.VMEM((CHUNK, 16), jnp.float32),
            pltpu.VMEM((CHUNK, 16), jnp.float32),
            pltpu.VMEM((N_LANES,), jnp.float32),
            pltpu.VMEM((N_LANES,), jnp.int32),
            pltpu.VMEM((N_LANES,), jnp.float32),
            pltpu.VMEM((N_LANES,), jnp.float32),
            pltpu.VMEM((2 * N_LANES,), jnp.float32),
            pltpu.VMEM_SHARED((N_LANES,), jnp.float32),
            pltpu.SemaphoreType.DMA,
        ],
    )(ram16, idx)
    merged = jnp.maximum(parts[0], parts[1])
    return merged[:18].reshape(1, 18)


# speedup vs baseline: 246.9911x; 1.0016x over previous
import jax
import jax.numpy as jnp
import numpy as np
from jax import lax
from jax.experimental import pallas as pl
from jax.experimental.pallas import tpu as pltpu
from jax.experimental.pallas import tpu_sc as plsc

N_ROWS = 262144
N_CORES = 2
N_SUBCORES = 16
N_WORKERS = N_CORES * N_SUBCORES
ROWS_PER = N_ROWS // N_WORKERS
N_LANES = 16
CHUNK = 128
N_CHUNKS = ROWS_PER // CHUNK
NBUF = 16
UNROLL = 4
GROUPS = CHUNK // (N_LANES * UNROLL)
COL0 = 0

_IDX_NP = (np.arange(N_ROWS, dtype=np.int32) * 8 + 2).reshape(
    N_WORKERS, N_CHUNKS, CHUNK)


def _actions_bits(buf, base, lanes, accs):
    new_accs = []
    for u in range(UNROLL):
        rows = base + u * N_LANES + lanes
        c0 = jnp.zeros((N_LANES,), jnp.int32)
        mi_x = plsc.load_gather(buf, [rows, c0 + COL0])
        su_x = plsc.load_gather(buf, [rows, c0 + (COL0 + 1)])
        mi_y = plsc.load_gather(buf, [rows, c0 + (COL0 + 2)])
        su_y = plsc.load_gather(buf, [rows, c0 + (COL0 + 3)])

        dist_x = jnp.abs(su_x - mi_x)
        dist_y = jnp.abs(su_y - mi_y)
        go_down = su_y > mi_y
        go_up = su_y < mi_y
        go_right = su_x > mi_x
        go_left = su_x < mi_x
        cond_y = dist_y > 2.0
        cond_x_far = dist_x > 24.0
        cond_x_close = dist_x < 22.0

        def iv(v):
            return jnp.full((N_LANES,), v, jnp.int32)

        act_y = jnp.where(go_down, iv(5), iv(2))
        act_x_far = jnp.where(go_right, iv(3), iv(4))
        act_x_close = jnp.where(go_right, iv(4), iv(3))

        adf = iv(6)
        adf = jnp.where(go_up & go_left, iv(7), adf)
        adf = jnp.where(go_down & go_right, iv(8), adf)
        adf = jnp.where(go_down & go_left, iv(9), adf)
        adc = adf ^ 1

        action = jnp.where(
            cond_y,
            jnp.where(cond_x_close, adc, jnp.where(cond_x_far, adf, act_y)),
            jnp.where(cond_x_far, act_x_far,
                      jnp.where(cond_x_close, act_x_close, iv(1))))

        new_accs.append(accs[u] | lax.shift_left(iv(1), action))
    return tuple(new_accs)


def _sc_body(ram16_hbm, idx_hbm, out_hbm, idx_v, buf0, buf1, buf2, buf3,
             buf4, buf5, buf6, buf7, buf8, buf9, buf10, buf11, buf12, buf13,
             buf14, buf15, ind_v, sidx_v, z_v, acc_v, out_v, shared, sem):
    cid = lax.axis_index("c")
    sid = lax.axis_index("s")
    wid = sid * N_CORES + cid
    lanes = lax.iota(jnp.int32, N_LANES)

    pltpu.sync_copy(idx_hbm.at[wid], idx_v)

    bufs = (buf0, buf1, buf2, buf3, buf4, buf5, buf6, buf7,
            buf8, buf9, buf10, buf11, buf12, buf13, buf14, buf15)

    def issue(c, b):
        return pltpu.async_copy(ram16_hbm.at[idx_v.at[c]], bufs[b], sem)

    def wait(b):
        pltpu.make_async_copy(ram16_hbm.at[idx_v.at[0]], bufs[b], sem).wait()

    for b in range(NBUF):
        issue(b, b)

    zero = jnp.zeros((N_LANES,), jnp.int32)

    def compute(b, accs):
        for g in range(GROUPS):
            accs = _actions_bits(bufs[b], g * N_LANES * UNROLL, lanes, accs)
        return accs

    def body(i, accs):
        c = i * NBUF
        for b in range(NBUF):
            wait(b)
            accs = compute(b, accs)

            @pl.when(c + b + NBUF < N_CHUNKS)
            def _():
                issue(c + b + NBUF, b)

        return accs

    accs = lax.fori_loop(0, N_CHUNKS // NBUF, body, (zero,) * UNROLL)
    bits = accs[0]
    for u in range(1, UNROLL):
        bits = bits | accs[u]

    for s in (1, 2, 4, 8):
        rot = (lanes + s) & (N_LANES - 1)
        bits = bits | bits.at[rot].get(mode="promise_in_bounds")
    ind = lax.shift_right_logical(bits, lanes) & 1
    ind_v[...] = ind.astype(jnp.float32)
    sidx_v[...] = lanes
    z_v[...] = jnp.zeros((N_LANES,), jnp.float32)

    @pl.when(sid == 0)
    def _init():
        pltpu.sync_copy(z_v, shared)

    plsc.subcore_barrier()
    pltpu.sync_copy(ind_v, shared.at[sidx_v], add=True)
    plsc.subcore_barrier()

    @pl.when(sid == 0)
    def _finalize():
        pltpu.sync_copy(shared, acc_v)
        seen = acc_v[...]
        out_v[pl.ds(0, N_LANES)] = jnp.where(
            seen > 0.0, jnp.full((N_LANES,), 1.0, jnp.float32),
            jnp.zeros((N_LANES,), jnp.float32))
        out_v[pl.ds(N_LANES, N_LANES)] = jnp.zeros((N_LANES,), jnp.float32)
        pltpu.sync_copy(out_v, out_hbm.at[cid])


@jax.jit
def kernel(ram):
    ram16 = ram.reshape(-1, 16)
    idx = jnp.asarray(_IDX_NP)
    mesh = plsc.VectorSubcoreMesh(
        core_axis_name="c", subcore_axis_name="s", num_cores=N_CORES)
    parts = pl.kernel(
        _sc_body,
        out_type=jax.ShapeDtypeStruct((N_CORES, 2 * N_LANES), jnp.float32),
        mesh=mesh,
        compiler_params=pltpu.CompilerParams(
            use_tc_tiling_on_sc=False, needs_layout_passes=False,
            disable_bounds_checks=True, disable_semaphore_checks=True,
            skip_device_barrier=True),
        scratch_types=[
            pltpu.VMEM((N_CHUNKS, CHUNK), jnp.int32),
            pltpu.VMEM((CHUNK, 16), jnp.float32),
            pltpu.VMEM((CHUNK, 16), jnp.float32),
            pltpu.VMEM((CHUNK, 16), jnp.float32),
            pltpu.VMEM((CHUNK, 16), jnp.float32),
            pltpu.VMEM((CHUNK, 16), jnp.float32),
            pltpu.VMEM((CHUNK, 16), jnp.float32),
            pltpu.VMEM((CHUNK, 16), jnp.float32),
            pltpu.VMEM((CHUNK, 16), jnp.float32),
            pltpu.VMEM((CHUNK, 16), jnp.float32),
            pltpu.VMEM((CHUNK, 16), jnp.float32),
            pltpu.VMEM((CHUNK, 16), jnp.float32),
            pltpu.VMEM((CHUNK, 16), jnp.float32),
            pltpu.VMEM((CHUNK, 16), jnp.float32),
            pltpu.VMEM((CHUNK, 16), jnp.float32),
            pltpu.VMEM((CHUNK, 16), jnp.float32),
            pltpu.VMEM((CHUNK, 16), jnp.float32),
            pltpu.VMEM((N_LANES,), jnp.float32),
            pltpu.VMEM((N_LANES,), jnp.int32),
            pltpu.VMEM((N_LANES,), jnp.float32),
            pltpu.VMEM((N_LANES,), jnp.float32),
            pltpu.VMEM((2 * N_LANES,), jnp.float32),
            pltpu.VMEM_SHARED((N_LANES,), jnp.float32),
            pltpu.SemaphoreType.DMA,
        ],
    )(ram16, idx)
    merged = jnp.maximum(parts[0], parts[1])
    return merged[:18].reshape(1, 18)
